# baseline (speedup 1.0000x reference)
import functools

import jax
import jax.numpy as jnp
from jax import lax
from jax.experimental import pallas as pl
from jax.experimental.pallas import tpu as pltpu
from jax.experimental.pallas import tpu_sc as plsc

_NC = 2
_NS = 16
_NW = _NC * _NS
_L = 16


def _sc_fm(table, gidx3d, B, F, D):
    CB = 128
    IW = 64
    RPC = CB * F
    G = RPC // IW
    b_per_w = B // _NW
    n_chunks = b_per_w // CB

    mesh = plsc.VectorSubcoreMesh(
        core_axis_name="c", subcore_axis_name="s",
        num_cores=_NC, num_subcores=_NS)

    @functools.partial(
        pl.kernel,
        mesh=mesh,
        out_type=jax.ShapeDtypeStruct((B, D), jnp.float32),
        scratch_types=[
            pltpu.VMEM((G, IW), jnp.int32),
            pltpu.VMEM((RPC, D), jnp.float32),
            pltpu.VMEM((CB, D), jnp.float32),
            pltpu.SemaphoreType.DMA,
        ],
        compiler_params=pltpu.CompilerParams(use_tc_tiling_on_sc=False),
    )
    def k(tab_hbm, idx_hbm, out_hbm, gidx_v, rows_v, fm_v, sem):
        wid = lax.axis_index("s") * _NC + lax.axis_index("c")

        def do_chunk(c, _):
            pltpu.sync_copy(idx_hbm.at[wid * n_chunks + c], gidx_v)
            cps = [
                pltpu.async_copy(tab_hbm.at[gidx_v.at[g]],
                                 rows_v.at[pl.ds(g * IW, IW)], sem)
                for g in range(G)
            ]
            for cp in cps:
                cp.wait()

            def body(b, _):
                r0 = rows_v[b * F]
                s1 = r0
                s2 = r0 * r0
                for f in range(1, F):
                    r = rows_v[b * F + f]
                    s1 = s1 + r
                    s2 = s2 + r * r
                fm_v[b] = 0.5 * (s1 * s1 - s2)
                return 0

            lax.fori_loop(0, CB, body, 0)
            pltpu.sync_copy(fm_v,
                            out_hbm.at[pl.ds((wid * n_chunks + c) * CB, CB)])
            return 0

        lax.fori_loop(0, n_chunks, do_chunk, 0)

    return k(table, gidx3d)


def _tc_dense(x, fm, W1, b1, W2, b2, W3a, W3b, b3):
    B, F = x.shape
    D = fm.shape[1]
    H = W1.shape[1]
    C = W3a.shape[1]
    BB = 512
    grid = (B // BB,)

    def body(x_ref, fm_ref, w1_ref, b1_ref, w2_ref, b2_ref, w3a_ref,
             w3b_ref, b3_ref, o_ref):
        h = jnp.maximum(
            jnp.dot(x_ref[...], w1_ref[...],
                    preferred_element_type=jnp.float32,
                    precision=lax.Precision.HIGHEST) + b1_ref[...], 0.0)
        h = jnp.maximum(
            jnp.dot(h, w2_ref[...], preferred_element_type=jnp.float32,
                    precision=lax.Precision.HIGHEST) + b2_ref[...], 0.0)
        o_ref[...] = (
            jnp.dot(fm_ref[...], w3a_ref[...],
                    preferred_element_type=jnp.float32,
                    precision=lax.Precision.HIGHEST)
            + jnp.dot(h, w3b_ref[...], preferred_element_type=jnp.float32,
                      precision=lax.Precision.HIGHEST)
            + b3_ref[...])

    full = lambda s: pl.BlockSpec(s, lambda i: (0,) * len(s))
    return pl.pallas_call(
        body,
        grid=grid,
        in_specs=[
            pl.BlockSpec((BB, F), lambda i: (i, 0)),
            pl.BlockSpec((BB, D), lambda i: (i, 0)),
            full((F, H)), full((1, H)), full((H, H)), full((1, H)),
            full((D, C)), full((H, C)), full((1, C)),
        ],
        out_specs=pl.BlockSpec((BB, C), lambda i: (i, 0)),
        out_shape=jax.ShapeDtypeStruct((B, C), jnp.float32),
    )(x, fm, W1, b1.reshape(1, H), W2, b2.reshape(1, H),
      W3a, W3b, b3.reshape(1, C))


def kernel(x, tables, W1, b1, W2, b2, W3, b3):
    B, F = x.shape
    _, V, D = tables.shape
    CB = 128
    idx = x.astype(jnp.int32)
    gidx = idx + (jnp.arange(F, dtype=jnp.int32) * V)[None, :]
    gidx3d = gidx.reshape(B // CB, (CB * F) // 64, 64)
    table = tables.reshape(F * V, D)
    fm = _sc_fm(table, gidx3d, B, F, D)
    return _tc_dense(x, fm, W1, b1, W2, b2, W3[:D], W3[D:], b3)
